# baseline (speedup 1.0000x reference)
import jax
import jax.numpy as jnp
from jax.experimental import pallas as pl
from jax.experimental.pallas import tpu as pltpu

_EPS = 0.1
_KSEL = 34


def _u32(v):
    return jnp.uint32(v)


def _kth_kernel(st_ref, z_ref, out_ref, h_ref, l_ref):
    d = st_ref.shape[0]
    lanes = out_ref.shape[-1]
    nchunk = 8
    c = d // nchunk
    for i in range(nchunk):
        sl = pl.ds(i * c, c)
        x = st_ref[sl, :] + _EPS * z_ref[sl, :]
        ub = jax.lax.bitcast_convert_type(x, jnp.uint32)
        flip = jnp.where(ub >= _u32(0x80000000), _u32(0xFFFFFFFF),
                         _u32(0x80000000))
        u = ub ^ flip
        h_ref[sl, :] = ((u >> 16) ^ _u32(0x8000)).astype(jnp.int16)
        l_ref[sl, :] = ((u & _u32(0xFFFF)) ^ _u32(0x8000)).astype(jnp.int16)

    def _tree(m):
        mb = jnp.where(m, jnp.bfloat16(1), jnp.bfloat16(0))
        grp = 64 if mb.shape[0] % 64 == 0 else mb.shape[0]
        g = jnp.sum(mb.reshape(mb.shape[0] // grp, grp, lanes), axis=1,
                    dtype=jnp.bfloat16)
        return jnp.sum(g.astype(jnp.float32), axis=0).reshape(1, lanes)

    def _count(m):
        return _tree(m)

    def _s16(trial):
        return (trial ^ _u32(0x8000)).astype(jnp.int16)

    def _pass2(j, prefix, rank):
        b1 = 1 << (15 - 2 * j)
        b0 = 1 << (14 - 2 * j)
        t1 = _s16(prefix | _u32(b0))
        t2 = _s16(prefix | _u32(b1))
        t3 = _s16(prefix | _u32(b1 | b0))
        c1 = jnp.zeros((1, lanes), jnp.float32)
        c2 = jnp.zeros((1, lanes), jnp.float32)
        c3 = jnp.zeros((1, lanes), jnp.float32)
        for i in range(nchunk):
            hv = h_ref[pl.ds(i * c, c), :]
            c1 = c1 + _tree(hv >= t1)
            c2 = c2 + _tree(hv >= t2)
            c3 = c3 + _tree(hv >= t3)
        hi = c2 >= rank
        lo = jnp.where(hi, c3, c1) >= rank
        prefix = prefix | jnp.where(hi, _u32(b1), _u32(0))
        return prefix | jnp.where(lo, _u32(b0), _u32(0))

    ksel_f = jnp.full((1, lanes), float(_KSEL), jnp.float32)
    p16 = jnp.zeros((1, lanes), jnp.uint32)
    for j in range(8):
        p16 = _pass2(j, p16, ksel_f)
    p16s = (p16 ^ _u32(0x8000)).astype(jnp.int16)

    cgt = _count(h_ref[...] > p16s)
    jrank = float(_KSEL) - cgt

    for i in range(nchunk):
        sl = pl.ds(i * c, c)
        h_ref[sl, :] = jnp.where(h_ref[sl, :] == p16s, l_ref[sl, :],
                                 jnp.int16(-32768))

    plo = jnp.zeros((1, lanes), jnp.uint32)
    for j in range(8):
        plo = _pass2(j, plo, jrank)

    key = (p16 << 16) | plo
    flip2 = jnp.where(key >= _u32(0x80000000), _u32(0x80000000),
                      _u32(0xFFFFFFFF))
    out_ref[...] = jax.lax.bitcast_convert_type(key ^ flip2,
                                                jnp.float32)[None]


def _loss_kernel(kth_ref, s_ref, y_ref, out_ref):
    kth = kth_ref[...]
    skp1 = jnp.mean(kth, axis=1, keepdims=True)
    y = y_ref[...]
    s = s_ref[...]
    iot = jax.lax.broadcasted_iota(jnp.int32, s.shape, 1)
    correct = jnp.sum(jnp.where(iot == y, s, 0.0), axis=1,
                      keepdims=True)
    num = jnp.maximum(1.0 + skp1 - correct, 0.0)
    out_ref[...] = (jnp.sum(num) / s.shape[0]).reshape(1, 1)


def kernel(s, y, Z):
    B, D = s.shape
    N = Z.shape[2]
    R = B * N
    LB = 128
    nblk = R // LB
    reps = LB // B

    zt = jnp.transpose(Z, (1, 2, 0)).reshape(D, R)
    st = jnp.concatenate([s.T] * reps, axis=1)

    kth_blocks = pl.pallas_call(
        _kth_kernel,
        grid=(nblk,),
        in_specs=[
            pl.BlockSpec((D, LB), lambda i: (0, 0)),
            pl.BlockSpec((D, LB), lambda i: (0, i)),
        ],
        out_specs=pl.BlockSpec((1, 1, LB), lambda i: (i, 0, 0)),
        out_shape=jax.ShapeDtypeStruct((nblk, 1, LB), jnp.float32),
        scratch_shapes=[pltpu.VMEM((D, LB), jnp.int16),
                        pltpu.VMEM((D, LB), jnp.int16)],
    )(st, zt)

    kth_bn = kth_blocks.reshape(N, B).T

    loss = pl.pallas_call(
        _loss_kernel,
        out_specs=pl.BlockSpec((1, 1), lambda: (0, 0)),
        out_shape=jax.ShapeDtypeStruct((1, 1), jnp.float32),
    )(kth_bn, s, y.reshape(B, 1).astype(jnp.int32))
    return loss[0, 0]


# speedup vs baseline: 2.0376x; 2.0376x over previous
import jax
import jax.numpy as jnp
from jax.experimental import pallas as pl
from jax.experimental.pallas import tpu as pltpu

_EPS = 0.1
_KSEL = 34


def _u32(v):
    return jnp.uint32(v)


def _kth_kernel(st_ref, z_ref, out_ref, h_ref, l_ref):
    d = st_ref.shape[0]
    lanes = out_ref.shape[-1]
    nchunk = 8
    c = d // nchunk
    for i in range(nchunk):
        sl = pl.ds(i * c, c)
        x = st_ref[sl, :] + _EPS * z_ref[sl, :]
        ub = jax.lax.bitcast_convert_type(x, jnp.uint32)
        flip = jnp.where(ub >= _u32(0x80000000), _u32(0xFFFFFFFF),
                         _u32(0x80000000))
        u = ub ^ flip
        h_ref[sl, :] = ((u >> 16) ^ _u32(0x8000)).astype(jnp.int16)
        l_ref[sl, :] = ((u & _u32(0xFFFF)) ^ _u32(0x8000)).astype(jnp.int16)

    ones_bf = jnp.ones((1, d), jnp.bfloat16)

    def _count(m):
        mb = jnp.where(m, jnp.bfloat16(1), jnp.bfloat16(0))
        return jax.lax.dot_general(
            ones_bf, mb, (((1,), (0,)), ((), ())),
            preferred_element_type=jnp.float32)

    def _count_ge(t16):
        return _count(h_ref[...] >= t16)

    def body_a(i, prefix):
        trial = prefix | (_u32(0x8000) >> i)
        t16 = (trial ^ _u32(0x8000)).astype(jnp.int16)
        return jnp.where(_count_ge(t16) >= float(_KSEL), trial, prefix)

    p16 = jnp.zeros((1, lanes), jnp.uint32)
    for i in range(16):
        p16 = body_a(i, p16)
    p16s = (p16 ^ _u32(0x8000)).astype(jnp.int16)

    cgt = _count(h_ref[...] > p16s)
    jrank = float(_KSEL) - cgt

    for i in range(nchunk):
        sl = pl.ds(i * c, c)
        h_ref[sl, :] = jnp.where(h_ref[sl, :] == p16s, l_ref[sl, :],
                                 jnp.int16(-32768))

    def body_b(i, prefix):
        trial = prefix | (_u32(0x8000) >> i)
        t16 = (trial ^ _u32(0x8000)).astype(jnp.int16)
        return jnp.where(_count_ge(t16) >= jrank, trial, prefix)

    plo = jnp.zeros((1, lanes), jnp.uint32)
    for i in range(16):
        plo = body_b(i, plo)

    key = (p16 << 16) | plo
    flip2 = jnp.where(key >= _u32(0x80000000), _u32(0x80000000),
                      _u32(0xFFFFFFFF))
    out_ref[...] = jax.lax.bitcast_convert_type(key ^ flip2,
                                                jnp.float32)[None]


def _loss_kernel(kth_ref, s_ref, y_ref, out_ref):
    kth = kth_ref[...]
    skp1 = jnp.mean(kth, axis=1, keepdims=True)
    y = y_ref[...]
    s = s_ref[...]
    iot = jax.lax.broadcasted_iota(jnp.int32, s.shape, 1)
    correct = jnp.sum(jnp.where(iot == y, s, 0.0), axis=1,
                      keepdims=True)
    num = jnp.maximum(1.0 + skp1 - correct, 0.0)
    out_ref[...] = (jnp.sum(num) / s.shape[0]).reshape(1, 1)


def kernel(s, y, Z):
    B, D = s.shape
    N = Z.shape[2]
    R = B * N
    LB = 128
    nblk = R // LB
    reps = LB // B

    zt = jnp.transpose(Z, (1, 2, 0)).reshape(D, R)
    st = jnp.concatenate([s.T] * reps, axis=1)

    kth_blocks = pl.pallas_call(
        _kth_kernel,
        grid=(nblk,),
        in_specs=[
            pl.BlockSpec((D, LB), lambda i: (0, 0)),
            pl.BlockSpec((D, LB), lambda i: (0, i)),
        ],
        out_specs=pl.BlockSpec((1, 1, LB), lambda i: (i, 0, 0)),
        out_shape=jax.ShapeDtypeStruct((nblk, 1, LB), jnp.float32),
        scratch_shapes=[pltpu.VMEM((D, LB), jnp.int16),
                        pltpu.VMEM((D, LB), jnp.int16)],
    )(st, zt)

    kth_bn = kth_blocks.reshape(N, B).T

    loss = pl.pallas_call(
        _loss_kernel,
        out_specs=pl.BlockSpec((1, 1), lambda: (0, 0)),
        out_shape=jax.ShapeDtypeStruct((1, 1), jnp.float32),
    )(kth_bn, s, y.reshape(B, 1).astype(jnp.int32))
    return loss[0, 0]
